# baseline (speedup 1.0000x reference)
import functools
import math

import jax
import jax.numpy as jnp
from jax import lax
from jax.experimental import pallas as pl
from jax.experimental.pallas import tpu as pltpu
from jax.experimental.pallas import tpu_sc as plsc

D = 128
A = 4
G = 16
_CHUNK = 128
_PREC = jax.lax.Precision.HIGHEST


def _silu(v):
    return v * jax.nn.sigmoid(v)


def _contract(y, ea, scale):
    acc = ea[:, 0:1] * y[:, 0:D]
    for j in range(1, A):
        acc = acc + ea[:, j:j + 1] * y[:, j * D:(j + 1) * D]
    return acc * scale


def _sc_counts():
    try:
        info = plsc.get_sparse_core_info()
        return int(info.num_cores), int(info.num_subcores)
    except Exception:
        return 2, 16


def _embed_body(x_ref, a_ref, w_ref, o_ref):
    y = jnp.dot(x_ref[...], w_ref[...], preferred_element_type=jnp.float32, precision=_PREC)
    o_ref[...] = _contract(y, a_ref[...], 1.0 / math.sqrt(D * A))


def _embed(x, nattr, w_r):
    n = x.shape[0]
    bn = n // 16
    return pl.pallas_call(
        _embed_body,
        grid=(n // bn,),
        in_specs=[
            pl.BlockSpec((bn, D), lambda i: (i, 0)),
            pl.BlockSpec((bn, A), lambda i: (i, 0)),
            pl.BlockSpec((D, A * D), lambda i: (0, 0)),
        ],
        out_specs=pl.BlockSpec((bn, D), lambda i: (i, 0)),
        out_shape=jax.ShapeDtypeStruct((n, D), jnp.float32),
    )(x, nattr, w_r)


def _edge_body(inc_ref, outg_ref, ea_ref, w0a_ref, w0b_ref, w1_ref, o_ref):
    ea = ea_ref[...]
    y0 = jnp.dot(inc_ref[...], w0a_ref[...], preferred_element_type=jnp.float32, precision=_PREC)
    y0 = y0 + jnp.dot(outg_ref[...], w0b_ref[...],
                      preferred_element_type=jnp.float32, precision=_PREC)
    m = _silu(_contract(y0, ea, 1.0 / math.sqrt(2 * D * A)))
    y1 = jnp.dot(m, w1_ref[...], preferred_element_type=jnp.float32, precision=_PREC)
    o_ref[...] = _silu(_contract(y1, ea, 1.0 / math.sqrt(D * A)))


def _edge_mlp(inc, outg, ea, w0a, w0b, w1):
    ep = inc.shape[0]
    be = 2048
    return pl.pallas_call(
        _edge_body,
        grid=(ep // be,),
        in_specs=[
            pl.BlockSpec((be, D), lambda i: (i, 0)),
            pl.BlockSpec((be, D), lambda i: (i, 0)),
            pl.BlockSpec((be, A), lambda i: (i, 0)),
            pl.BlockSpec((D, A * D), lambda i: (0, 0)),
            pl.BlockSpec((D, A * D), lambda i: (0, 0)),
            pl.BlockSpec((D, A * D), lambda i: (0, 0)),
        ],
        out_specs=pl.BlockSpec((be, D), lambda i: (i, 0)),
        out_shape=jax.ShapeDtypeStruct((ep, D), jnp.float32),
    )(inc, outg, ea, w0a, w0b, w1)


def _update_body(nd_ref, a0_ref, a1_ref, na_ref, w0a_ref, w0b_ref, w1_ref,
                 o_ref):
    nd = nd_ref[...]
    agg = a0_ref[0] + a1_ref[0]
    na = na_ref[...]
    y0 = jnp.dot(nd, w0a_ref[...], preferred_element_type=jnp.float32, precision=_PREC)
    y0 = y0 + jnp.dot(agg, w0b_ref[...], preferred_element_type=jnp.float32, precision=_PREC)
    u = _silu(_contract(y0, na, 1.0 / math.sqrt(2 * D * A)))
    y1 = jnp.dot(u, w1_ref[...], preferred_element_type=jnp.float32, precision=_PREC)
    o_ref[...] = nd + _contract(y1, na, 1.0 / math.sqrt(D * A))


def _update(nodes, agg, nattr, w0a, w0b, w1):
    n = nodes.shape[0]
    bn = n // 16
    return pl.pallas_call(
        _update_body,
        grid=(n // bn,),
        in_specs=[
            pl.BlockSpec((bn, D), lambda i: (i, 0)),
            pl.BlockSpec((1, bn, D), lambda i: (0, i, 0)),
            pl.BlockSpec((1, bn, D), lambda i: (1, i, 0)),
            pl.BlockSpec((bn, A), lambda i: (i, 0)),
            pl.BlockSpec((D, A * D), lambda i: (0, 0)),
            pl.BlockSpec((D, A * D), lambda i: (0, 0)),
            pl.BlockSpec((D, A * D), lambda i: (0, 0)),
        ],
        out_specs=pl.BlockSpec((bn, D), lambda i: (i, 0)),
        out_shape=jax.ShapeDtypeStruct((n, D), jnp.float32),
    )(nodes, agg, agg, nattr, w0a, w0b, w1)


def _prepool_body(nd_ref, na_ref, w0_ref, w1_ref, o_ref):
    na = na_ref[...]
    y0 = jnp.dot(nd_ref[...], w0_ref[...], preferred_element_type=jnp.float32, precision=_PREC)
    h = _silu(_contract(y0, na, 1.0 / math.sqrt(D * A)))
    y1 = jnp.dot(h, w1_ref[...], preferred_element_type=jnp.float32, precision=_PREC)
    o_ref[...] = _contract(y1, na, 1.0 / math.sqrt(D * A))


def _prepool(nodes, nattr, w0, w1):
    n = nodes.shape[0]
    bn = n // 16
    return pl.pallas_call(
        _prepool_body,
        grid=(n // bn,),
        in_specs=[
            pl.BlockSpec((bn, D), lambda i: (i, 0)),
            pl.BlockSpec((bn, A), lambda i: (i, 0)),
            pl.BlockSpec((D, A * D), lambda i: (0, 0)),
            pl.BlockSpec((D, A * D), lambda i: (0, 0)),
        ],
        out_specs=pl.BlockSpec((bn, D), lambda i: (i, 0)),
        out_shape=jax.ShapeDtypeStruct((n, D), jnp.float32),
    )(nodes, nattr, w0, w1)


def _pool_body(h_ref, gi_ref, wpost_ref, wout_ref, o_ref, sums, cnt):
    i = pl.program_id(0)

    @pl.when(i == 0)
    def _():
        sums[...] = jnp.zeros_like(sums)
        cnt[...] = jnp.zeros_like(cnt)

    gi = gi_ref[...]
    bn = gi.shape[0]
    m = (gi == lax.broadcasted_iota(jnp.int32, (bn, G), 1)).astype(jnp.float32)
    h = h_ref[...]
    dn = (((0,), (0,)), ((), ()))
    sums[...] += lax.dot_general(m, h, dn, preferred_element_type=jnp.float32, precision=_PREC)
    cnt[...] += lax.dot_general(m, jnp.ones_like(h), dn,
                                preferred_element_type=jnp.float32, precision=_PREC)
    pooled = sums[...] / jnp.maximum(cnt[...], 1.0)
    h2 = _silu(jnp.dot(pooled, wpost_ref[...],
                       preferred_element_type=jnp.float32, precision=_PREC) / math.sqrt(D))
    o_ref[...] = jnp.dot(h2, wout_ref[...],
                         preferred_element_type=jnp.float32, precision=_PREC) / math.sqrt(D)


def _pool_decode(h, gi2d, wpost, wout):
    n = h.shape[0]
    bn = n // 16
    return pl.pallas_call(
        _pool_body,
        grid=(n // bn,),
        in_specs=[
            pl.BlockSpec((bn, D), lambda i: (i, 0)),
            pl.BlockSpec((bn, 1), lambda i: (i, 0)),
            pl.BlockSpec((D, D), lambda i: (0, 0)),
            pl.BlockSpec((D, 1), lambda i: (0, 0)),
        ],
        out_specs=pl.BlockSpec((G, 1), lambda i: (0, 0)),
        out_shape=jax.ShapeDtypeStruct((G, 1), jnp.float32),
        scratch_shapes=[
            pltpu.VMEM((G, D), jnp.float32),
            pltpu.VMEM((G, D), jnp.float32),
        ],
    )(h, gi2d, wpost, wout)


def _chunks_of(total, cap):
    out, off = [], 0
    while off < total:
        sz = min(cap, total - off)
        out.append((off, sz))
        off += sz
    return out


def _sc_gather(nodes, s_idx, r_idx):
    nc, ns = _sc_counts()
    nw = nc * ns
    ep = s_idx.shape[0]
    npad = nodes.shape[0]
    rt = npad // ns
    per_w = ep // nw
    ch = 64
    n2 = per_w // (2 * ch)
    stage_chunks = _chunks_of(rt, ch)
    mesh = plsc.VectorSubcoreMesh(core_axis_name="c", subcore_axis_name="s")
    out_t = (jax.ShapeDtypeStruct((ep, D), jnp.float32),
             jax.ShapeDtypeStruct((ep, D), jnp.float32))

    @functools.partial(
        pl.kernel, mesh=mesh, out_type=out_t,
        scratch_types=[
            pltpu.VMEM((per_w,), jnp.int32),
            pltpu.VMEM((per_w,), jnp.int32),
            pltpu.VMEM((2, ch, D), jnp.float32),
            pltpu.VMEM((2, ch, D), jnp.float32),
            pltpu.VMEM_SHARED((npad, D), jnp.float32),
            [pltpu.SemaphoreType.DMA] * 4,
            [pltpu.SemaphoreType.DMA] * 4,
        ],
    )
    def k(nodes_h, s_h, r_h, inc_h, outg_h, ix_s, ix_r, rw_s, rw_r, tbl,
          sg, sw):
        cid = lax.axis_index("c")
        sid = lax.axis_index("s")
        wid = sid * nc + cid
        base = wid * per_w
        row0 = sid * rt

        for coff, csz in stage_chunks:
            pltpu.sync_copy(nodes_h.at[pl.ds(row0 + coff, csz)],
                            rw_s.at[0, pl.ds(0, csz)])
            pltpu.sync_copy(rw_s.at[0, pl.ds(0, csz)],
                            tbl.at[pl.ds(row0 + coff, csz)])
        pltpu.sync_copy(s_h.at[pl.ds(base, per_w)], ix_s)
        pltpu.sync_copy(r_h.at[pl.ds(base, per_w)], ix_r)
        plsc.subcore_barrier()

        def gath(c, b):
            return (pltpu.async_copy(tbl.at[ix_s.at[pl.ds(c * ch, ch)]],
                                     rw_s.at[b], sg[b]),
                    pltpu.async_copy(tbl.at[ix_r.at[pl.ds(c * ch, ch)]],
                                     rw_r.at[b], sg[2 + b]))

        def wait_gath(c, b):
            pltpu.make_async_copy(tbl.at[ix_s.at[pl.ds(c * ch, ch)]],
                                  rw_s.at[b], sg[b]).wait()
            pltpu.make_async_copy(tbl.at[ix_r.at[pl.ds(c * ch, ch)]],
                                  rw_r.at[b], sg[2 + b]).wait()

        def write(c, b):
            off = base + c * ch
            return (pltpu.async_copy(rw_s.at[b], inc_h.at[pl.ds(off, ch)],
                                     sw[b]),
                    pltpu.async_copy(rw_r.at[b], outg_h.at[pl.ds(off, ch)],
                                     sw[2 + b]))

        def wait_write(c, b):
            off = base + c * ch
            pltpu.make_async_copy(rw_s.at[b], inc_h.at[pl.ds(off, ch)],
                                  sw[b]).wait()
            pltpu.make_async_copy(rw_r.at[b], outg_h.at[pl.ds(off, ch)],
                                  sw[2 + b]).wait()

        gath(0, 0)

        def body(j, _):
            c0 = 2 * j
            @pl.when(j > 0)
            def _():
                wait_write(c0 - 1, 1)

            gath(c0 + 1, 1)
            wait_gath(c0, 0)
            write(c0, 0)
            wait_gath(c0 + 1, 1)
            write(c0 + 1, 1)
            wait_write(c0, 0)

            @pl.when(j < n2 - 1)
            def _():
                gath(c0 + 2, 0)

            return 0

        lax.fori_loop(0, n2, body, 0, unroll=False)
        wait_write(2 * n2 - 1, 1)

    return k(nodes, s_idx, r_idx)


def _sc_scatter(msg, r_idx, nrow):
    nc, ns = _sc_counts()
    nw = nc * ns
    ep = msg.shape[0]
    per_w = ep // nw
    n_ch = per_w // _CHUNK
    rows_t = nrow // ns
    mesh = plsc.VectorSubcoreMesh(core_axis_name="c", subcore_axis_name="s")
    out_t = jax.ShapeDtypeStruct((nc, nrow, D), jnp.float32)

    chunks = _chunks_of(rows_t, _CHUNK)

    @functools.partial(
        pl.kernel, mesh=mesh, out_type=out_t,
        scratch_types=[
            pltpu.VMEM((_CHUNK,), jnp.int32),
            pltpu.VMEM((_CHUNK, D), jnp.float32),
            pltpu.VMEM_SHARED((nrow, D), jnp.float32),
            pltpu.SemaphoreType.DMA,
        ],
    )
    def k(msg_h, r_h, out_h, ix, rw, acc, sem):
        cid = lax.axis_index("c")
        sid = lax.axis_index("s")
        wid = sid * nc + cid
        base = wid * per_w
        row0 = sid * rows_t

        def zr(r, _):
            def zc(c, __):
                rw[r, pl.ds(c * 16, 16)] = jnp.zeros((16,), jnp.float32)
                return 0
            lax.fori_loop(0, D // 16, zc, 0, unroll=True)
            return 0

        lax.fori_loop(0, _CHUNK, zr, 0, unroll=False)
        for coff, csz in chunks:
            pltpu.sync_copy(rw.at[pl.ds(0, csz)],
                            acc.at[pl.ds(row0 + coff, csz)])
        plsc.subcore_barrier()

        def body(i, _):
            off = base + i * _CHUNK
            pltpu.sync_copy(r_h.at[pl.ds(off, _CHUNK)], ix)
            pltpu.sync_copy(msg_h.at[pl.ds(off, _CHUNK)], rw)
            pltpu.sync_copy(rw, acc.at[ix], add=True)
            return 0

        lax.fori_loop(0, n_ch, body, 0, unroll=False)
        plsc.subcore_barrier()

        for coff, csz in chunks:
            pltpu.sync_copy(acc.at[pl.ds(row0 + coff, csz)],
                            rw.at[pl.ds(0, csz)])
            pltpu.sync_copy(rw.at[pl.ds(0, csz)],
                            out_h.at[cid, pl.ds(row0 + coff, csz)])

    return k(msg, r_idx)


def kernel(x, node_attr, edge_attr, edge_index, graph_idx, W_embed, W_msg0,
           W_msg1, W_upd0, W_upd1, W_pre0, W_pre1, W_post0, W_out):
    n, d = x.shape
    e = edge_index.shape[1]
    nc, ns = _sc_counts()
    nw = nc * ns
    quant = nw * _CHUNK
    ep = ((e + quant - 1) // quant) * quant
    pad = ep - e
    npad = ((n + 1 + 127) // 128) * 128
    np_ = npad - n

    senders = edge_index[0].astype(jnp.int32)
    receivers = edge_index[1].astype(jnp.int32)
    zpad = jnp.zeros((pad,), jnp.int32)
    s_p = jnp.concatenate([senders, zpad])
    r_p = jnp.concatenate([receivers, zpad])
    r_scat = jnp.concatenate([receivers, jnp.full((pad,), n, jnp.int32)])
    ea_p = jnp.concatenate(
        [edge_attr, jnp.zeros((pad, A), jnp.float32)], axis=0)
    x_p = jnp.concatenate([x, jnp.zeros((np_, d), jnp.float32)])
    nattr_p = jnp.concatenate([node_attr, jnp.zeros((np_, A), jnp.float32)])

    w_embed_r = W_embed.reshape(D, A * D)
    nodes = _embed(x_p, nattr_p, w_embed_r)

    num_layers = W_msg0.shape[0]
    for l in range(num_layers):
        w0a = W_msg0[l, :D].reshape(D, A * D)
        w0b = W_msg0[l, D:].reshape(D, A * D)
        w1 = W_msg1[l].reshape(D, A * D)
        u0a = W_upd0[l, :D].reshape(D, A * D)
        u0b = W_upd0[l, D:].reshape(D, A * D)
        u1 = W_upd1[l].reshape(D, A * D)

        inc, outg = _sc_gather(nodes, s_p, r_p)
        msg = _edge_mlp(inc, outg, ea_p, w0a, w0b, w1)
        agg = _sc_scatter(msg, r_scat, npad)
        nodes = _update(nodes, agg, nattr_p, u0a, u0b, u1)

    h = _prepool(nodes, nattr_p, W_pre0.reshape(D, A * D),
                 W_pre1.reshape(D, A * D))
    gi_p = jnp.concatenate(
        [graph_idx.astype(jnp.int32), jnp.full((np_,), G, jnp.int32)])
    out = _pool_decode(h, gi_p.reshape(npad, 1), W_post0, W_out)
    return out.reshape(G)


# speedup vs baseline: 1.7191x; 1.7191x over previous
import functools
import math

import jax
import jax.numpy as jnp
from jax import lax
from jax.experimental import pallas as pl
from jax.experimental.pallas import tpu as pltpu
from jax.experimental.pallas import tpu_sc as plsc

D = 128
A = 4
G = 16
_CHUNK = 128
_PREC = jax.lax.Precision.HIGHEST


def _dot3(x, w):
    xh = x.astype(jnp.bfloat16)
    xl = (x - xh.astype(jnp.float32)).astype(jnp.bfloat16)
    wh = w.astype(jnp.bfloat16)
    wl = (w - wh.astype(jnp.float32)).astype(jnp.bfloat16)
    f32 = jnp.float32
    return (jnp.dot(xh, wh, preferred_element_type=f32)
            + (jnp.dot(xh, wl, preferred_element_type=f32)
               + jnp.dot(xl, wh, preferred_element_type=f32)))


def _silu(v):
    return v * jax.nn.sigmoid(v)


def _contract(y, ea, scale):
    acc = ea[:, 0:1] * y[:, 0:D]
    for j in range(1, A):
        acc = acc + ea[:, j:j + 1] * y[:, j * D:(j + 1) * D]
    return acc * scale


def _sc_counts():
    try:
        info = plsc.get_sparse_core_info()
        return int(info.num_cores), int(info.num_subcores)
    except Exception:
        return 2, 16


def _embed_body(x_ref, a_ref, w_ref, o_ref):
    y = _dot3(x_ref[...], w_ref[...])
    o_ref[...] = _contract(y, a_ref[...], 1.0 / math.sqrt(D * A))


def _embed(x, nattr, w_r):
    n = x.shape[0]
    bn = n // 16
    return pl.pallas_call(
        _embed_body,
        grid=(n // bn,),
        in_specs=[
            pl.BlockSpec((bn, D), lambda i: (i, 0)),
            pl.BlockSpec((bn, A), lambda i: (i, 0)),
            pl.BlockSpec((D, A * D), lambda i: (0, 0)),
        ],
        out_specs=pl.BlockSpec((bn, D), lambda i: (i, 0)),
        out_shape=jax.ShapeDtypeStruct((n, D), jnp.float32),
    )(x, nattr, w_r)


def _edge_body(inc_ref, outg_ref, ea_ref, w0a_ref, w0b_ref, w1_ref, o_ref):
    ea = ea_ref[...]
    y0 = _dot3(inc_ref[...], w0a_ref[...])
    y0 = y0 + _dot3(outg_ref[...], w0b_ref[...])
    m = _silu(_contract(y0, ea, 1.0 / math.sqrt(2 * D * A)))
    y1 = _dot3(m, w1_ref[...])
    o_ref[...] = _silu(_contract(y1, ea, 1.0 / math.sqrt(D * A)))


def _edge_mlp(inc, outg, ea, w0a, w0b, w1):
    ep = inc.shape[0]
    be = 2048
    return pl.pallas_call(
        _edge_body,
        grid=(ep // be,),
        in_specs=[
            pl.BlockSpec((be, D), lambda i: (i, 0)),
            pl.BlockSpec((be, D), lambda i: (i, 0)),
            pl.BlockSpec((be, A), lambda i: (i, 0)),
            pl.BlockSpec((D, A * D), lambda i: (0, 0)),
            pl.BlockSpec((D, A * D), lambda i: (0, 0)),
            pl.BlockSpec((D, A * D), lambda i: (0, 0)),
        ],
        out_specs=pl.BlockSpec((be, D), lambda i: (i, 0)),
        out_shape=jax.ShapeDtypeStruct((ep, D), jnp.float32),
    )(inc, outg, ea, w0a, w0b, w1)


def _update_body(nd_ref, a0_ref, a1_ref, na_ref, w0a_ref, w0b_ref, w1_ref,
                 o_ref):
    nd = nd_ref[...]
    agg = a0_ref[0] + a1_ref[0]
    na = na_ref[...]
    y0 = _dot3(nd, w0a_ref[...])
    y0 = y0 + _dot3(agg, w0b_ref[...])
    u = _silu(_contract(y0, na, 1.0 / math.sqrt(2 * D * A)))
    y1 = _dot3(u, w1_ref[...])
    o_ref[...] = nd + _contract(y1, na, 1.0 / math.sqrt(D * A))


def _update(nodes, agg, nattr, w0a, w0b, w1):
    n = nodes.shape[0]
    bn = n // 16
    return pl.pallas_call(
        _update_body,
        grid=(n // bn,),
        in_specs=[
            pl.BlockSpec((bn, D), lambda i: (i, 0)),
            pl.BlockSpec((1, bn, D), lambda i: (0, i, 0)),
            pl.BlockSpec((1, bn, D), lambda i: (1, i, 0)),
            pl.BlockSpec((bn, A), lambda i: (i, 0)),
            pl.BlockSpec((D, A * D), lambda i: (0, 0)),
            pl.BlockSpec((D, A * D), lambda i: (0, 0)),
            pl.BlockSpec((D, A * D), lambda i: (0, 0)),
        ],
        out_specs=pl.BlockSpec((bn, D), lambda i: (i, 0)),
        out_shape=jax.ShapeDtypeStruct((n, D), jnp.float32),
    )(nodes, agg, agg, nattr, w0a, w0b, w1)


def _prepool_body(nd_ref, na_ref, w0_ref, w1_ref, o_ref):
    na = na_ref[...]
    y0 = _dot3(nd_ref[...], w0_ref[...])
    h = _silu(_contract(y0, na, 1.0 / math.sqrt(D * A)))
    y1 = _dot3(h, w1_ref[...])
    o_ref[...] = _contract(y1, na, 1.0 / math.sqrt(D * A))


def _prepool(nodes, nattr, w0, w1):
    n = nodes.shape[0]
    bn = n // 16
    return pl.pallas_call(
        _prepool_body,
        grid=(n // bn,),
        in_specs=[
            pl.BlockSpec((bn, D), lambda i: (i, 0)),
            pl.BlockSpec((bn, A), lambda i: (i, 0)),
            pl.BlockSpec((D, A * D), lambda i: (0, 0)),
            pl.BlockSpec((D, A * D), lambda i: (0, 0)),
        ],
        out_specs=pl.BlockSpec((bn, D), lambda i: (i, 0)),
        out_shape=jax.ShapeDtypeStruct((n, D), jnp.float32),
    )(nodes, nattr, w0, w1)


def _pool_body(h_ref, gi_ref, wpost_ref, wout_ref, o_ref, sums, cnt):
    i = pl.program_id(0)

    @pl.when(i == 0)
    def _():
        sums[...] = jnp.zeros_like(sums)
        cnt[...] = jnp.zeros_like(cnt)

    gi = gi_ref[...]
    bn = gi.shape[0]
    m = (gi == lax.broadcasted_iota(jnp.int32, (bn, G), 1)).astype(jnp.float32)
    h = h_ref[...]
    dn = (((0,), (0,)), ((), ()))
    sums[...] += lax.dot_general(m, h, dn, preferred_element_type=jnp.float32, precision=_PREC)
    cnt[...] += lax.dot_general(m, jnp.ones_like(h), dn,
                                preferred_element_type=jnp.float32, precision=_PREC)
    pooled = sums[...] / jnp.maximum(cnt[...], 1.0)
    h2 = _silu(jnp.dot(pooled, wpost_ref[...],
                       preferred_element_type=jnp.float32, precision=_PREC) / math.sqrt(D))
    o_ref[...] = jnp.dot(h2, wout_ref[...],
                         preferred_element_type=jnp.float32, precision=_PREC) / math.sqrt(D)


def _pool_decode(h, gi2d, wpost, wout):
    n = h.shape[0]
    bn = n // 16
    return pl.pallas_call(
        _pool_body,
        grid=(n // bn,),
        in_specs=[
            pl.BlockSpec((bn, D), lambda i: (i, 0)),
            pl.BlockSpec((bn, 1), lambda i: (i, 0)),
            pl.BlockSpec((D, D), lambda i: (0, 0)),
            pl.BlockSpec((D, 1), lambda i: (0, 0)),
        ],
        out_specs=pl.BlockSpec((G, 1), lambda i: (0, 0)),
        out_shape=jax.ShapeDtypeStruct((G, 1), jnp.float32),
        scratch_shapes=[
            pltpu.VMEM((G, D), jnp.float32),
            pltpu.VMEM((G, D), jnp.float32),
        ],
    )(h, gi2d, wpost, wout)


def _chunks_of(total, cap):
    out, off = [], 0
    while off < total:
        sz = min(cap, total - off)
        out.append((off, sz))
        off += sz
    return out


def _sc_gather(nodes, s_idx, r_idx):
    nc, ns = _sc_counts()
    nw = nc * ns
    ep = s_idx.shape[0]
    npad = nodes.shape[0]
    rt = npad // ns
    per_w = ep // nw
    ch = 64
    n2 = per_w // (2 * ch)
    stage_chunks = _chunks_of(rt, ch)
    mesh = plsc.VectorSubcoreMesh(core_axis_name="c", subcore_axis_name="s")
    out_t = (jax.ShapeDtypeStruct((ep, D), jnp.float32),
             jax.ShapeDtypeStruct((ep, D), jnp.float32))

    @functools.partial(
        pl.kernel, mesh=mesh, out_type=out_t,
        scratch_types=[
            pltpu.VMEM((per_w,), jnp.int32),
            pltpu.VMEM((per_w,), jnp.int32),
            pltpu.VMEM((2, ch, D), jnp.float32),
            pltpu.VMEM((2, ch, D), jnp.float32),
            pltpu.VMEM_SHARED((npad, D), jnp.float32),
            [pltpu.SemaphoreType.DMA] * 4,
            [pltpu.SemaphoreType.DMA] * 4,
        ],
    )
    def k(nodes_h, s_h, r_h, inc_h, outg_h, ix_s, ix_r, rw_s, rw_r, tbl,
          sg, sw):
        cid = lax.axis_index("c")
        sid = lax.axis_index("s")
        wid = sid * nc + cid
        base = wid * per_w
        row0 = sid * rt

        for coff, csz in stage_chunks:
            pltpu.sync_copy(nodes_h.at[pl.ds(row0 + coff, csz)],
                            rw_s.at[0, pl.ds(0, csz)])
            pltpu.sync_copy(rw_s.at[0, pl.ds(0, csz)],
                            tbl.at[pl.ds(row0 + coff, csz)])
        pltpu.sync_copy(s_h.at[pl.ds(base, per_w)], ix_s)
        pltpu.sync_copy(r_h.at[pl.ds(base, per_w)], ix_r)
        plsc.subcore_barrier()

        def gath(c, b):
            return (pltpu.async_copy(tbl.at[ix_s.at[pl.ds(c * ch, ch)]],
                                     rw_s.at[b], sg[b]),
                    pltpu.async_copy(tbl.at[ix_r.at[pl.ds(c * ch, ch)]],
                                     rw_r.at[b], sg[2 + b]))

        def wait_gath(c, b):
            pltpu.make_async_copy(tbl.at[ix_s.at[pl.ds(c * ch, ch)]],
                                  rw_s.at[b], sg[b]).wait()
            pltpu.make_async_copy(tbl.at[ix_r.at[pl.ds(c * ch, ch)]],
                                  rw_r.at[b], sg[2 + b]).wait()

        def write(c, b):
            off = base + c * ch
            return (pltpu.async_copy(rw_s.at[b], inc_h.at[pl.ds(off, ch)],
                                     sw[b]),
                    pltpu.async_copy(rw_r.at[b], outg_h.at[pl.ds(off, ch)],
                                     sw[2 + b]))

        def wait_write(c, b):
            off = base + c * ch
            pltpu.make_async_copy(rw_s.at[b], inc_h.at[pl.ds(off, ch)],
                                  sw[b]).wait()
            pltpu.make_async_copy(rw_r.at[b], outg_h.at[pl.ds(off, ch)],
                                  sw[2 + b]).wait()

        gath(0, 0)

        def body(j, _):
            c0 = 2 * j
            @pl.when(j > 0)
            def _():
                wait_write(c0 - 1, 1)

            gath(c0 + 1, 1)
            wait_gath(c0, 0)
            write(c0, 0)
            wait_gath(c0 + 1, 1)
            write(c0 + 1, 1)
            wait_write(c0, 0)

            @pl.when(j < n2 - 1)
            def _():
                gath(c0 + 2, 0)

            return 0

        lax.fori_loop(0, n2, body, 0, unroll=False)
        wait_write(2 * n2 - 1, 1)

    return k(nodes, s_idx, r_idx)


def _sc_scatter(msg, r_idx, nrow):
    nc, ns = _sc_counts()
    nw = nc * ns
    ep = msg.shape[0]
    per_w = ep // nw
    n_ch = per_w // _CHUNK
    rows_t = nrow // ns
    mesh = plsc.VectorSubcoreMesh(core_axis_name="c", subcore_axis_name="s")
    out_t = jax.ShapeDtypeStruct((nc, nrow, D), jnp.float32)

    chunks = _chunks_of(rows_t, _CHUNK)

    n2 = n_ch // 2

    @functools.partial(
        pl.kernel, mesh=mesh, out_type=out_t,
        scratch_types=[
            pltpu.VMEM((2, _CHUNK), jnp.int32),
            pltpu.VMEM((2, _CHUNK, D), jnp.float32),
            pltpu.VMEM_SHARED((nrow, D), jnp.float32),
            [pltpu.SemaphoreType.DMA] * 2,
            [pltpu.SemaphoreType.DMA] * 2,
        ],
    )
    def k(msg_h, r_h, out_h, ix2, rw2, acc, si, sm):
        cid = lax.axis_index("c")
        sid = lax.axis_index("s")
        wid = sid * nc + cid
        base = wid * per_w
        row0 = sid * rows_t

        def load(c, b):
            off = base + c * _CHUNK
            pltpu.async_copy(r_h.at[pl.ds(off, _CHUNK)], ix2.at[b], si[b])
            pltpu.async_copy(msg_h.at[pl.ds(off, _CHUNK)], rw2.at[b], sm[b])

        def wait_load(c, b):
            off = base + c * _CHUNK
            pltpu.make_async_copy(r_h.at[pl.ds(off, _CHUNK)], ix2.at[b],
                                  si[b]).wait()
            pltpu.make_async_copy(msg_h.at[pl.ds(off, _CHUNK)], rw2.at[b],
                                  sm[b]).wait()

        load(0, 0)

        def zr(r, _):
            def zc(c, __):
                rw2[1, r, pl.ds(c * 16, 16)] = jnp.zeros((16,), jnp.float32)
                return 0
            lax.fori_loop(0, D // 16, zc, 0, unroll=True)
            return 0

        lax.fori_loop(0, _CHUNK, zr, 0, unroll=False)
        for coff, csz in chunks:
            pltpu.sync_copy(rw2.at[1, pl.ds(0, csz)],
                            acc.at[pl.ds(row0 + coff, csz)])
        plsc.subcore_barrier()

        def body(j, _):
            c0 = 2 * j
            load(c0 + 1, 1)
            wait_load(c0, 0)
            pltpu.sync_copy(rw2.at[0], acc.at[ix2.at[0]], add=True)

            @pl.when(j < n2 - 1)
            def _():
                load(c0 + 2, 0)

            wait_load(c0 + 1, 1)
            pltpu.sync_copy(rw2.at[1], acc.at[ix2.at[1]], add=True)
            return 0

        lax.fori_loop(0, n2, body, 0, unroll=False)
        plsc.subcore_barrier()

        for coff, csz in chunks:
            pltpu.sync_copy(acc.at[pl.ds(row0 + coff, csz)],
                            rw2.at[0, pl.ds(0, csz)])
            pltpu.sync_copy(rw2.at[0, pl.ds(0, csz)],
                            out_h.at[cid, pl.ds(row0 + coff, csz)])

    return k(msg, r_idx)


def kernel(x, node_attr, edge_attr, edge_index, graph_idx, W_embed, W_msg0,
           W_msg1, W_upd0, W_upd1, W_pre0, W_pre1, W_post0, W_out):
    n, d = x.shape
    e = edge_index.shape[1]
    nc, ns = _sc_counts()
    nw = nc * ns
    quant = nw * _CHUNK
    ep = ((e + quant - 1) // quant) * quant
    pad = ep - e
    npad = ((n + 1 + 127) // 128) * 128
    np_ = npad - n

    senders = edge_index[0].astype(jnp.int32)
    receivers = edge_index[1].astype(jnp.int32)
    zpad = jnp.zeros((pad,), jnp.int32)
    s_p = jnp.concatenate([senders, zpad])
    r_p = jnp.concatenate([receivers, zpad])
    r_scat = jnp.concatenate([receivers, jnp.full((pad,), n, jnp.int32)])
    ea_p = jnp.concatenate(
        [edge_attr, jnp.zeros((pad, A), jnp.float32)], axis=0)
    x_p = jnp.concatenate([x, jnp.zeros((np_, d), jnp.float32)])
    nattr_p = jnp.concatenate([node_attr, jnp.zeros((np_, A), jnp.float32)])

    w_embed_r = W_embed.reshape(D, A * D)
    nodes = _embed(x_p, nattr_p, w_embed_r)

    num_layers = W_msg0.shape[0]
    for l in range(num_layers):
        w0a = W_msg0[l, :D].reshape(D, A * D)
        w0b = W_msg0[l, D:].reshape(D, A * D)
        w1 = W_msg1[l].reshape(D, A * D)
        u0a = W_upd0[l, :D].reshape(D, A * D)
        u0b = W_upd0[l, D:].reshape(D, A * D)
        u1 = W_upd1[l].reshape(D, A * D)

        inc, outg = _sc_gather(nodes, s_p, r_p)
        msg = _edge_mlp(inc, outg, ea_p, w0a, w0b, w1)
        agg = _sc_scatter(msg, r_scat, npad)
        nodes = _update(nodes, agg, nattr_p, u0a, u0b, u1)

    h = _prepool(nodes, nattr_p, W_pre0.reshape(D, A * D),
                 W_pre1.reshape(D, A * D))
    gi_p = jnp.concatenate(
        [graph_idx.astype(jnp.int32), jnp.full((np_,), G, jnp.int32)])
    out = _pool_decode(h, gi_p.reshape(npad, 1), W_post0, W_out)
    return out.reshape(G)


# speedup vs baseline: 1.7276x; 1.0049x over previous
import functools
import math

import jax
import jax.numpy as jnp
from jax import lax
from jax.experimental import pallas as pl
from jax.experimental.pallas import tpu as pltpu
from jax.experimental.pallas import tpu_sc as plsc

D = 128
A = 4
G = 16
_CHUNK = 128
_PREC = jax.lax.Precision.HIGHEST


def _split_lohi(w):
    wh = w.astype(jnp.bfloat16)
    wl = (w - wh.astype(jnp.float32)).astype(jnp.bfloat16)
    return wh, wl


def _dot3(x, wh, wl):
    xh = x.astype(jnp.bfloat16)
    xl = (x - xh.astype(jnp.float32)).astype(jnp.bfloat16)
    f32 = jnp.float32
    return (jnp.dot(xh, wh, preferred_element_type=f32)
            + (jnp.dot(xh, wl, preferred_element_type=f32)
               + jnp.dot(xl, wh, preferred_element_type=f32)))


def _wspec(n=1):
    return [pl.BlockSpec((D, A * D), lambda i: (0, 0)) for _ in range(2 * n)]


def _silu(v):
    return v * jax.nn.sigmoid(v)


def _contract(y, ea, scale):
    acc = ea[:, 0:1] * y[:, 0:D]
    for j in range(1, A):
        acc = acc + ea[:, j:j + 1] * y[:, j * D:(j + 1) * D]
    return acc * scale


def _sc_counts():
    try:
        info = plsc.get_sparse_core_info()
        return int(info.num_cores), int(info.num_subcores)
    except Exception:
        return 2, 16


def _embed_body(x_ref, a_ref, wh_ref, wl_ref, o_ref):
    y = _dot3(x_ref[...], wh_ref[...], wl_ref[...])
    o_ref[...] = _contract(y, a_ref[...], 1.0 / math.sqrt(D * A))


def _embed(x, nattr, w_r):
    n = x.shape[0]
    bn = n // 16
    return pl.pallas_call(
        _embed_body,
        grid=(n // bn,),
        in_specs=[
            pl.BlockSpec((bn, D), lambda i: (i, 0)),
            pl.BlockSpec((bn, A), lambda i: (i, 0)),
        ] + _wspec(1),
        out_specs=pl.BlockSpec((bn, D), lambda i: (i, 0)),
        out_shape=jax.ShapeDtypeStruct((n, D), jnp.float32),
    )(x, nattr, *_split_lohi(w_r))


def _edge_body(inc_ref, outg_ref, ea_ref, w0ah, w0al, w0bh, w0bl, w1h, w1l,
               o_ref):
    ea = ea_ref[...]
    y0 = _dot3(inc_ref[...], w0ah[...], w0al[...])
    y0 = y0 + _dot3(outg_ref[...], w0bh[...], w0bl[...])
    m = _silu(_contract(y0, ea, 1.0 / math.sqrt(2 * D * A)))
    y1 = _dot3(m, w1h[...], w1l[...])
    o_ref[...] = _silu(_contract(y1, ea, 1.0 / math.sqrt(D * A)))


def _edge_mlp(inc, outg, ea, w0a, w0b, w1):
    ep = inc.shape[0]
    be = 2048
    return pl.pallas_call(
        _edge_body,
        grid=(ep // be,),
        in_specs=[
            pl.BlockSpec((be, D), lambda i: (i, 0)),
            pl.BlockSpec((be, D), lambda i: (i, 0)),
            pl.BlockSpec((be, A), lambda i: (i, 0)),
        ] + _wspec(3),
        out_specs=pl.BlockSpec((be, D), lambda i: (i, 0)),
        out_shape=jax.ShapeDtypeStruct((ep, D), jnp.float32),
    )(inc, outg, ea, *_split_lohi(w0a), *_split_lohi(w0b), *_split_lohi(w1))


def _update_body(nd_ref, a0_ref, a1_ref, na_ref, w0ah, w0al, w0bh, w0bl,
                 w1h, w1l, o_ref):
    nd = nd_ref[...]
    agg = a0_ref[0] + a1_ref[0]
    na = na_ref[...]
    y0 = _dot3(nd, w0ah[...], w0al[...])
    y0 = y0 + _dot3(agg, w0bh[...], w0bl[...])
    u = _silu(_contract(y0, na, 1.0 / math.sqrt(2 * D * A)))
    y1 = _dot3(u, w1h[...], w1l[...])
    o_ref[...] = nd + _contract(y1, na, 1.0 / math.sqrt(D * A))


def _update(nodes, agg, nattr, w0a, w0b, w1):
    n = nodes.shape[0]
    bn = n // 16
    return pl.pallas_call(
        _update_body,
        grid=(n // bn,),
        in_specs=[
            pl.BlockSpec((bn, D), lambda i: (i, 0)),
            pl.BlockSpec((1, bn, D), lambda i: (0, i, 0)),
            pl.BlockSpec((1, bn, D), lambda i: (1, i, 0)),
            pl.BlockSpec((bn, A), lambda i: (i, 0)),
        ] + _wspec(3),
        out_specs=pl.BlockSpec((bn, D), lambda i: (i, 0)),
        out_shape=jax.ShapeDtypeStruct((n, D), jnp.float32),
    )(nodes, agg, agg, nattr, *_split_lohi(w0a), *_split_lohi(w0b),
      *_split_lohi(w1))


def _prepool_body(nd_ref, na_ref, w0h, w0l, w1h, w1l, o_ref):
    na = na_ref[...]
    y0 = _dot3(nd_ref[...], w0h[...], w0l[...])
    h = _silu(_contract(y0, na, 1.0 / math.sqrt(D * A)))
    y1 = _dot3(h, w1h[...], w1l[...])
    o_ref[...] = _contract(y1, na, 1.0 / math.sqrt(D * A))


def _prepool(nodes, nattr, w0, w1):
    n = nodes.shape[0]
    bn = n // 16
    return pl.pallas_call(
        _prepool_body,
        grid=(n // bn,),
        in_specs=[
            pl.BlockSpec((bn, D), lambda i: (i, 0)),
            pl.BlockSpec((bn, A), lambda i: (i, 0)),
        ] + _wspec(2),
        out_specs=pl.BlockSpec((bn, D), lambda i: (i, 0)),
        out_shape=jax.ShapeDtypeStruct((n, D), jnp.float32),
    )(nodes, nattr, *_split_lohi(w0), *_split_lohi(w1))


def _pool_body(h_ref, gi_ref, wpost_ref, wout_ref, o_ref, sums, cnt):
    i = pl.program_id(0)

    @pl.when(i == 0)
    def _():
        sums[...] = jnp.zeros_like(sums)
        cnt[...] = jnp.zeros_like(cnt)

    gi = gi_ref[...]
    bn = gi.shape[0]
    m = (gi == lax.broadcasted_iota(jnp.int32, (bn, G), 1)).astype(jnp.float32)
    h = h_ref[...]
    dn = (((0,), (0,)), ((), ()))
    sums[...] += lax.dot_general(m, h, dn, preferred_element_type=jnp.float32, precision=_PREC)
    cnt[...] += lax.dot_general(m, jnp.ones_like(h), dn,
                                preferred_element_type=jnp.float32, precision=_PREC)
    pooled = sums[...] / jnp.maximum(cnt[...], 1.0)
    h2 = _silu(jnp.dot(pooled, wpost_ref[...],
                       preferred_element_type=jnp.float32, precision=_PREC) / math.sqrt(D))
    o_ref[...] = jnp.dot(h2, wout_ref[...],
                         preferred_element_type=jnp.float32, precision=_PREC) / math.sqrt(D)


def _pool_decode(h, gi2d, wpost, wout):
    n = h.shape[0]
    bn = n // 16
    return pl.pallas_call(
        _pool_body,
        grid=(n // bn,),
        in_specs=[
            pl.BlockSpec((bn, D), lambda i: (i, 0)),
            pl.BlockSpec((bn, 1), lambda i: (i, 0)),
            pl.BlockSpec((D, D), lambda i: (0, 0)),
            pl.BlockSpec((D, 1), lambda i: (0, 0)),
        ],
        out_specs=pl.BlockSpec((G, 1), lambda i: (0, 0)),
        out_shape=jax.ShapeDtypeStruct((G, 1), jnp.float32),
        scratch_shapes=[
            pltpu.VMEM((G, D), jnp.float32),
            pltpu.VMEM((G, D), jnp.float32),
        ],
    )(h, gi2d, wpost, wout)


def _chunks_of(total, cap):
    out, off = [], 0
    while off < total:
        sz = min(cap, total - off)
        out.append((off, sz))
        off += sz
    return out


def _sc_gather(nodes, s_idx, r_idx):
    nc, ns = _sc_counts()
    nw = nc * ns
    ep = s_idx.shape[0]
    npad = nodes.shape[0]
    rt = npad // ns
    per_w = ep // nw
    ch = 64
    n2 = per_w // (2 * ch)
    stage_chunks = _chunks_of(rt, ch)
    mesh = plsc.VectorSubcoreMesh(core_axis_name="c", subcore_axis_name="s")
    out_t = (jax.ShapeDtypeStruct((ep, D), jnp.float32),
             jax.ShapeDtypeStruct((ep, D), jnp.float32))

    @functools.partial(
        pl.kernel, mesh=mesh, out_type=out_t,
        scratch_types=[
            pltpu.VMEM((per_w,), jnp.int32),
            pltpu.VMEM((per_w,), jnp.int32),
            pltpu.VMEM((2, ch, D), jnp.float32),
            pltpu.VMEM((2, ch, D), jnp.float32),
            pltpu.VMEM_SHARED((npad, D), jnp.float32),
            [pltpu.SemaphoreType.DMA] * 4,
            [pltpu.SemaphoreType.DMA] * 4,
        ],
    )
    def k(nodes_h, s_h, r_h, inc_h, outg_h, ix_s, ix_r, rw_s, rw_r, tbl,
          sg, sw):
        cid = lax.axis_index("c")
        sid = lax.axis_index("s")
        wid = sid * nc + cid
        base = wid * per_w
        row0 = sid * rt

        for coff, csz in stage_chunks:
            pltpu.sync_copy(nodes_h.at[pl.ds(row0 + coff, csz)],
                            rw_s.at[0, pl.ds(0, csz)])
            pltpu.sync_copy(rw_s.at[0, pl.ds(0, csz)],
                            tbl.at[pl.ds(row0 + coff, csz)])
        pltpu.sync_copy(s_h.at[pl.ds(base, per_w)], ix_s)
        pltpu.sync_copy(r_h.at[pl.ds(base, per_w)], ix_r)
        plsc.subcore_barrier()

        def gath(c, b):
            return (pltpu.async_copy(tbl.at[ix_s.at[pl.ds(c * ch, ch)]],
                                     rw_s.at[b], sg[b]),
                    pltpu.async_copy(tbl.at[ix_r.at[pl.ds(c * ch, ch)]],
                                     rw_r.at[b], sg[2 + b]))

        def wait_gath(c, b):
            pltpu.make_async_copy(tbl.at[ix_s.at[pl.ds(c * ch, ch)]],
                                  rw_s.at[b], sg[b]).wait()
            pltpu.make_async_copy(tbl.at[ix_r.at[pl.ds(c * ch, ch)]],
                                  rw_r.at[b], sg[2 + b]).wait()

        def write(c, b):
            off = base + c * ch
            return (pltpu.async_copy(rw_s.at[b], inc_h.at[pl.ds(off, ch)],
                                     sw[b]),
                    pltpu.async_copy(rw_r.at[b], outg_h.at[pl.ds(off, ch)],
                                     sw[2 + b]))

        def wait_write(c, b):
            off = base + c * ch
            pltpu.make_async_copy(rw_s.at[b], inc_h.at[pl.ds(off, ch)],
                                  sw[b]).wait()
            pltpu.make_async_copy(rw_r.at[b], outg_h.at[pl.ds(off, ch)],
                                  sw[2 + b]).wait()

        gath(0, 0)

        def body(j, _):
            c0 = 2 * j
            @pl.when(j > 0)
            def _():
                wait_write(c0 - 1, 1)

            gath(c0 + 1, 1)
            wait_gath(c0, 0)
            write(c0, 0)
            wait_gath(c0 + 1, 1)
            write(c0 + 1, 1)
            wait_write(c0, 0)

            @pl.when(j < n2 - 1)
            def _():
                gath(c0 + 2, 0)

            return 0

        lax.fori_loop(0, n2, body, 0, unroll=False)
        wait_write(2 * n2 - 1, 1)

    return k(nodes, s_idx, r_idx)


def _sc_scatter(msg, r_idx, nrow):
    nc, ns = _sc_counts()
    nw = nc * ns
    ep = msg.shape[0]
    per_w = ep // nw
    n_ch = per_w // _CHUNK
    rows_t = nrow // ns
    mesh = plsc.VectorSubcoreMesh(core_axis_name="c", subcore_axis_name="s")
    out_t = jax.ShapeDtypeStruct((nc, nrow, D), jnp.float32)

    chunks = _chunks_of(rows_t, _CHUNK)

    n2 = n_ch // 2

    @functools.partial(
        pl.kernel, mesh=mesh, out_type=out_t,
        scratch_types=[
            pltpu.VMEM((2, _CHUNK), jnp.int32),
            pltpu.VMEM((2, _CHUNK, D), jnp.float32),
            pltpu.VMEM_SHARED((nrow, D), jnp.float32),
            [pltpu.SemaphoreType.DMA] * 2,
            [pltpu.SemaphoreType.DMA] * 2,
        ],
    )
    def k(msg_h, r_h, out_h, ix2, rw2, acc, si, sm):
        cid = lax.axis_index("c")
        sid = lax.axis_index("s")
        wid = sid * nc + cid
        base = wid * per_w
        row0 = sid * rows_t

        def load(c, b):
            off = base + c * _CHUNK
            pltpu.async_copy(r_h.at[pl.ds(off, _CHUNK)], ix2.at[b], si[b])
            pltpu.async_copy(msg_h.at[pl.ds(off, _CHUNK)], rw2.at[b], sm[b])

        def wait_load(c, b):
            off = base + c * _CHUNK
            pltpu.make_async_copy(r_h.at[pl.ds(off, _CHUNK)], ix2.at[b],
                                  si[b]).wait()
            pltpu.make_async_copy(msg_h.at[pl.ds(off, _CHUNK)], rw2.at[b],
                                  sm[b]).wait()

        load(0, 0)

        def zr(r, _):
            def zc(c, __):
                rw2[1, r, pl.ds(c * 16, 16)] = jnp.zeros((16,), jnp.float32)
                return 0
            lax.fori_loop(0, D // 16, zc, 0, unroll=True)
            return 0

        lax.fori_loop(0, _CHUNK, zr, 0, unroll=False)
        for coff, csz in chunks:
            pltpu.sync_copy(rw2.at[1, pl.ds(0, csz)],
                            acc.at[pl.ds(row0 + coff, csz)])
        plsc.subcore_barrier()

        def body(j, _):
            c0 = 2 * j
            load(c0 + 1, 1)
            wait_load(c0, 0)
            pltpu.sync_copy(rw2.at[0], acc.at[ix2.at[0]], add=True)

            @pl.when(j < n2 - 1)
            def _():
                load(c0 + 2, 0)

            wait_load(c0 + 1, 1)
            pltpu.sync_copy(rw2.at[1], acc.at[ix2.at[1]], add=True)
            return 0

        lax.fori_loop(0, n2, body, 0, unroll=False)
        plsc.subcore_barrier()

        for coff, csz in chunks:
            pltpu.sync_copy(acc.at[pl.ds(row0 + coff, csz)],
                            rw2.at[0, pl.ds(0, csz)])
            pltpu.sync_copy(rw2.at[0, pl.ds(0, csz)],
                            out_h.at[cid, pl.ds(row0 + coff, csz)])

    return k(msg, r_idx)


def kernel(x, node_attr, edge_attr, edge_index, graph_idx, W_embed, W_msg0,
           W_msg1, W_upd0, W_upd1, W_pre0, W_pre1, W_post0, W_out):
    n, d = x.shape
    e = edge_index.shape[1]
    nc, ns = _sc_counts()
    nw = nc * ns
    quant = nw * _CHUNK
    ep = ((e + quant - 1) // quant) * quant
    pad = ep - e
    npad = ((n + 1 + 127) // 128) * 128
    np_ = npad - n

    senders = edge_index[0].astype(jnp.int32)
    receivers = edge_index[1].astype(jnp.int32)
    zpad = jnp.zeros((pad,), jnp.int32)
    s_p = jnp.concatenate([senders, zpad])
    r_p = jnp.concatenate([receivers, zpad])
    r_scat = jnp.concatenate([receivers, jnp.full((pad,), n, jnp.int32)])
    ea_p = jnp.concatenate(
        [edge_attr, jnp.zeros((pad, A), jnp.float32)], axis=0)
    x_p = jnp.concatenate([x, jnp.zeros((np_, d), jnp.float32)])
    nattr_p = jnp.concatenate([node_attr, jnp.zeros((np_, A), jnp.float32)])

    w_embed_r = W_embed.reshape(D, A * D)
    nodes = _embed(x_p, nattr_p, w_embed_r)

    num_layers = W_msg0.shape[0]
    for l in range(num_layers):
        w0a = W_msg0[l, :D].reshape(D, A * D)
        w0b = W_msg0[l, D:].reshape(D, A * D)
        w1 = W_msg1[l].reshape(D, A * D)
        u0a = W_upd0[l, :D].reshape(D, A * D)
        u0b = W_upd0[l, D:].reshape(D, A * D)
        u1 = W_upd1[l].reshape(D, A * D)

        inc, outg = _sc_gather(nodes, s_p, r_p)
        msg = _edge_mlp(inc, outg, ea_p, w0a, w0b, w1)
        agg = _sc_scatter(msg, r_scat, npad)
        nodes = _update(nodes, agg, nattr_p, u0a, u0b, u1)

    h = _prepool(nodes, nattr_p, W_pre0.reshape(D, A * D),
                 W_pre1.reshape(D, A * D))
    gi_p = jnp.concatenate(
        [graph_idx.astype(jnp.int32), jnp.full((np_,), G, jnp.int32)])
    out = _pool_decode(h, gi_p.reshape(npad, 1), W_post0, W_out)
    return out.reshape(G)


# speedup vs baseline: 1.7934x; 1.0381x over previous
import functools
import math

import jax
import jax.numpy as jnp
from jax import lax
from jax.experimental import pallas as pl
from jax.experimental.pallas import tpu as pltpu
from jax.experimental.pallas import tpu_sc as plsc

D = 128
A = 4
G = 16
_CHUNK = 128
_PREC = jax.lax.Precision.HIGHEST


def _split_lohi(w):
    wh = w.astype(jnp.bfloat16)
    wl = (w - wh.astype(jnp.float32)).astype(jnp.bfloat16)
    return wh, wl


def _dot3(x, wh, wl):
    xh = x.astype(jnp.bfloat16)
    xl = (x - xh.astype(jnp.float32)).astype(jnp.bfloat16)
    f32 = jnp.float32
    return (jnp.dot(xh, wh, preferred_element_type=f32)
            + (jnp.dot(xh, wl, preferred_element_type=f32)
               + jnp.dot(xl, wh, preferred_element_type=f32)))


def _wspec(n=1):
    return [pl.BlockSpec((D, A * D), lambda i: (0, 0)) for _ in range(2 * n)]


def _silu(v):
    return v * jax.nn.sigmoid(v)


def _contract(y, ea, scale):
    acc = ea[:, 0:1] * y[:, 0:D]
    for j in range(1, A):
        acc = acc + ea[:, j:j + 1] * y[:, j * D:(j + 1) * D]
    return acc * scale


def _sc_counts():
    try:
        info = plsc.get_sparse_core_info()
        return int(info.num_cores), int(info.num_subcores)
    except Exception:
        return 2, 16


def _embed_body(x_ref, a_ref, wh_ref, wl_ref, o_ref):
    y = _dot3(x_ref[...], wh_ref[...], wl_ref[...])
    o_ref[...] = _contract(y, a_ref[...], 1.0 / math.sqrt(D * A))


def _embed(x, nattr, w_r):
    n = x.shape[0]
    bn = n // 16
    return pl.pallas_call(
        _embed_body,
        grid=(n // bn,),
        in_specs=[
            pl.BlockSpec((bn, D), lambda i: (i, 0)),
            pl.BlockSpec((bn, A), lambda i: (i, 0)),
        ] + _wspec(1),
        out_specs=pl.BlockSpec((bn, D), lambda i: (i, 0)),
        out_shape=jax.ShapeDtypeStruct((n, D), jnp.float32),
    )(x, nattr, *_split_lohi(w_r))


def _edge_body(inc_ref, outg_ref, ea_ref, w0ah, w0al, w0bh, w0bl, w1h, w1l,
               o_ref):
    ea = ea_ref[...]
    y0 = _dot3(inc_ref[...], w0ah[...], w0al[...])
    y0 = y0 + _dot3(outg_ref[...], w0bh[...], w0bl[...])
    m = _silu(_contract(y0, ea, 1.0 / math.sqrt(2 * D * A)))
    y1 = _dot3(m, w1h[...], w1l[...])
    o_ref[...] = _silu(_contract(y1, ea, 1.0 / math.sqrt(D * A)))


def _edge_mlp(inc, outg, ea, w0a, w0b, w1):
    ep = inc.shape[0]
    be = 2048
    return pl.pallas_call(
        _edge_body,
        grid=(ep // be,),
        in_specs=[
            pl.BlockSpec((be, D), lambda i: (i, 0)),
            pl.BlockSpec((be, D), lambda i: (i, 0)),
            pl.BlockSpec((be, A), lambda i: (i, 0)),
        ] + _wspec(3),
        out_specs=pl.BlockSpec((be, D), lambda i: (i, 0)),
        out_shape=jax.ShapeDtypeStruct((ep, D), jnp.float32),
    )(inc, outg, ea, *_split_lohi(w0a), *_split_lohi(w0b), *_split_lohi(w1))


def _update_body(nd_ref, a0_ref, a1_ref, a2_ref, a3_ref, na_ref, w0ah, w0al,
                 w0bh, w0bl, w1h, w1l, o_ref):
    nd = nd_ref[...]
    agg = (a0_ref[0] + a1_ref[0]) + (a2_ref[0] + a3_ref[0])
    na = na_ref[...]
    y0 = _dot3(nd, w0ah[...], w0al[...])
    y0 = y0 + _dot3(agg, w0bh[...], w0bl[...])
    u = _silu(_contract(y0, na, 1.0 / math.sqrt(2 * D * A)))
    y1 = _dot3(u, w1h[...], w1l[...])
    o_ref[...] = nd + _contract(y1, na, 1.0 / math.sqrt(D * A))


def _update(nodes, agg_a, agg_b, nattr, w0a, w0b, w1):
    n = nodes.shape[0]
    bn = n // 16
    return pl.pallas_call(
        _update_body,
        grid=(n // bn,),
        in_specs=[
            pl.BlockSpec((bn, D), lambda i: (i, 0)),
            pl.BlockSpec((1, bn, D), lambda i: (0, i, 0)),
            pl.BlockSpec((1, bn, D), lambda i: (1, i, 0)),
            pl.BlockSpec((1, bn, D), lambda i: (0, i, 0)),
            pl.BlockSpec((1, bn, D), lambda i: (1, i, 0)),
            pl.BlockSpec((bn, A), lambda i: (i, 0)),
        ] + _wspec(3),
        out_specs=pl.BlockSpec((bn, D), lambda i: (i, 0)),
        out_shape=jax.ShapeDtypeStruct((n, D), jnp.float32),
    )(nodes, agg_a, agg_a, agg_b, agg_b, nattr, *_split_lohi(w0a),
      *_split_lohi(w0b), *_split_lohi(w1))


def _prepool_body(nd_ref, na_ref, w0h, w0l, w1h, w1l, o_ref):
    na = na_ref[...]
    y0 = _dot3(nd_ref[...], w0h[...], w0l[...])
    h = _silu(_contract(y0, na, 1.0 / math.sqrt(D * A)))
    y1 = _dot3(h, w1h[...], w1l[...])
    o_ref[...] = _contract(y1, na, 1.0 / math.sqrt(D * A))


def _prepool(nodes, nattr, w0, w1):
    n = nodes.shape[0]
    bn = n // 16
    return pl.pallas_call(
        _prepool_body,
        grid=(n // bn,),
        in_specs=[
            pl.BlockSpec((bn, D), lambda i: (i, 0)),
            pl.BlockSpec((bn, A), lambda i: (i, 0)),
        ] + _wspec(2),
        out_specs=pl.BlockSpec((bn, D), lambda i: (i, 0)),
        out_shape=jax.ShapeDtypeStruct((n, D), jnp.float32),
    )(nodes, nattr, *_split_lohi(w0), *_split_lohi(w1))


def _pool_body(h_ref, gi_ref, wpost_ref, wout_ref, o_ref, sums, cnt):
    i = pl.program_id(0)

    @pl.when(i == 0)
    def _():
        sums[...] = jnp.zeros_like(sums)
        cnt[...] = jnp.zeros_like(cnt)

    gi = gi_ref[...]
    bn = gi.shape[0]
    m = (gi == lax.broadcasted_iota(jnp.int32, (bn, G), 1)).astype(jnp.float32)
    h = h_ref[...]
    dn = (((0,), (0,)), ((), ()))
    sums[...] += lax.dot_general(m, h, dn, preferred_element_type=jnp.float32, precision=_PREC)
    cnt[...] += lax.dot_general(m, jnp.ones_like(h), dn,
                                preferred_element_type=jnp.float32, precision=_PREC)
    pooled = sums[...] / jnp.maximum(cnt[...], 1.0)
    h2 = _silu(jnp.dot(pooled, wpost_ref[...],
                       preferred_element_type=jnp.float32, precision=_PREC) / math.sqrt(D))
    o_ref[...] = jnp.dot(h2, wout_ref[...],
                         preferred_element_type=jnp.float32, precision=_PREC) / math.sqrt(D)


def _pool_decode(h, gi2d, wpost, wout):
    n = h.shape[0]
    bn = n // 16
    return pl.pallas_call(
        _pool_body,
        grid=(n // bn,),
        in_specs=[
            pl.BlockSpec((bn, D), lambda i: (i, 0)),
            pl.BlockSpec((bn, 1), lambda i: (i, 0)),
            pl.BlockSpec((D, D), lambda i: (0, 0)),
            pl.BlockSpec((D, 1), lambda i: (0, 0)),
        ],
        out_specs=pl.BlockSpec((G, 1), lambda i: (0, 0)),
        out_shape=jax.ShapeDtypeStruct((G, 1), jnp.float32),
        scratch_shapes=[
            pltpu.VMEM((G, D), jnp.float32),
            pltpu.VMEM((G, D), jnp.float32),
        ],
    )(h, gi2d, wpost, wout)


def _chunks_of(total, cap):
    out, off = [], 0
    while off < total:
        sz = min(cap, total - off)
        out.append((off, sz))
        off += sz
    return out


def _sc_gather(nodes, s_idx, r_idx):
    nc, ns = _sc_counts()
    nw = nc * ns
    ep = s_idx.shape[0]
    npad = nodes.shape[0]
    rt = npad // ns
    per_w = ep // nw
    ch = 64
    n2 = per_w // (2 * ch)
    stage_chunks = _chunks_of(rt, ch)
    mesh = plsc.VectorSubcoreMesh(core_axis_name="c", subcore_axis_name="s")
    out_t = (jax.ShapeDtypeStruct((ep, D), jnp.float32),
             jax.ShapeDtypeStruct((ep, D), jnp.float32))

    @functools.partial(
        pl.kernel, mesh=mesh, out_type=out_t,
        scratch_types=[
            pltpu.VMEM((per_w,), jnp.int32),
            pltpu.VMEM((per_w,), jnp.int32),
            pltpu.VMEM((2, ch, D), jnp.float32),
            pltpu.VMEM((2, ch, D), jnp.float32),
            pltpu.VMEM_SHARED((npad, D), jnp.float32),
            [pltpu.SemaphoreType.DMA] * 4,
            [pltpu.SemaphoreType.DMA] * 4,
        ],
    )
    def k(nodes_h, s_h, r_h, inc_h, outg_h, ix_s, ix_r, rw_s, rw_r, tbl,
          sg, sw):
        cid = lax.axis_index("c")
        sid = lax.axis_index("s")
        wid = sid * nc + cid
        base = wid * per_w
        row0 = sid * rt

        for coff, csz in stage_chunks:
            pltpu.sync_copy(nodes_h.at[pl.ds(row0 + coff, csz)],
                            rw_s.at[0, pl.ds(0, csz)])
            pltpu.sync_copy(rw_s.at[0, pl.ds(0, csz)],
                            tbl.at[pl.ds(row0 + coff, csz)])
        pltpu.sync_copy(s_h.at[pl.ds(base, per_w)], ix_s)
        pltpu.sync_copy(r_h.at[pl.ds(base, per_w)], ix_r)
        plsc.subcore_barrier()

        def gath(c, b):
            return (pltpu.async_copy(tbl.at[ix_s.at[pl.ds(c * ch, ch)]],
                                     rw_s.at[b], sg[b]),
                    pltpu.async_copy(tbl.at[ix_r.at[pl.ds(c * ch, ch)]],
                                     rw_r.at[b], sg[2 + b]))

        def wait_gath(c, b):
            pltpu.make_async_copy(tbl.at[ix_s.at[pl.ds(c * ch, ch)]],
                                  rw_s.at[b], sg[b]).wait()
            pltpu.make_async_copy(tbl.at[ix_r.at[pl.ds(c * ch, ch)]],
                                  rw_r.at[b], sg[2 + b]).wait()

        def write(c, b):
            off = base + c * ch
            return (pltpu.async_copy(rw_s.at[b], inc_h.at[pl.ds(off, ch)],
                                     sw[b]),
                    pltpu.async_copy(rw_r.at[b], outg_h.at[pl.ds(off, ch)],
                                     sw[2 + b]))

        def wait_write(c, b):
            off = base + c * ch
            pltpu.make_async_copy(rw_s.at[b], inc_h.at[pl.ds(off, ch)],
                                  sw[b]).wait()
            pltpu.make_async_copy(rw_r.at[b], outg_h.at[pl.ds(off, ch)],
                                  sw[2 + b]).wait()

        gath(0, 0)

        def body(j, _):
            c0 = 2 * j
            @pl.when(j > 0)
            def _():
                wait_write(c0 - 1, 1)

            gath(c0 + 1, 1)
            wait_gath(c0, 0)
            write(c0, 0)
            wait_gath(c0 + 1, 1)
            write(c0 + 1, 1)
            wait_write(c0, 0)

            @pl.when(j < n2 - 1)
            def _():
                gath(c0 + 2, 0)

            return 0

        lax.fori_loop(0, n2, body, 0, unroll=False)
        wait_write(2 * n2 - 1, 1)

    return k(nodes, s_idx, r_idx)


def _sc_scatter(msg, r_idx, nrow):
    nc, ns = _sc_counts()
    nw = nc * ns
    ep = msg.shape[0]
    per_w = ep // nw
    n_ch = per_w // _CHUNK
    rows_t = nrow // ns
    mesh = plsc.VectorSubcoreMesh(core_axis_name="c", subcore_axis_name="s")
    out_t = jax.ShapeDtypeStruct((nc, nrow, D), jnp.float32)

    chunks = _chunks_of(rows_t, _CHUNK)

    n2 = n_ch // 2

    @functools.partial(
        pl.kernel, mesh=mesh, out_type=out_t,
        scratch_types=[
            pltpu.VMEM((2, _CHUNK), jnp.int32),
            pltpu.VMEM((2, _CHUNK, D), jnp.float32),
            pltpu.VMEM_SHARED((nrow, D), jnp.float32),
            [pltpu.SemaphoreType.DMA] * 2,
            [pltpu.SemaphoreType.DMA] * 2,
        ],
    )
    def k(msg_h, r_h, out_h, ix2, rw2, acc, si, sm):
        cid = lax.axis_index("c")
        sid = lax.axis_index("s")
        wid = sid * nc + cid
        base = wid * per_w
        row0 = sid * rows_t

        def load(c, b):
            off = base + c * _CHUNK
            pltpu.async_copy(r_h.at[pl.ds(off, _CHUNK)], ix2.at[b], si[b])
            pltpu.async_copy(msg_h.at[pl.ds(off, _CHUNK)], rw2.at[b], sm[b])

        def wait_load(c, b):
            off = base + c * _CHUNK
            pltpu.make_async_copy(r_h.at[pl.ds(off, _CHUNK)], ix2.at[b],
                                  si[b]).wait()
            pltpu.make_async_copy(msg_h.at[pl.ds(off, _CHUNK)], rw2.at[b],
                                  sm[b]).wait()

        load(0, 0)

        def zr(r, _):
            def zc(c, __):
                rw2[1, r, pl.ds(c * 16, 16)] = jnp.zeros((16,), jnp.float32)
                return 0
            lax.fori_loop(0, D // 16, zc, 0, unroll=True)
            return 0

        lax.fori_loop(0, _CHUNK, zr, 0, unroll=False)
        for coff, csz in chunks:
            pltpu.sync_copy(rw2.at[1, pl.ds(0, csz)],
                            acc.at[pl.ds(row0 + coff, csz)])
        plsc.subcore_barrier()

        def body(j, _):
            c0 = 2 * j
            load(c0 + 1, 1)
            wait_load(c0, 0)
            pltpu.sync_copy(rw2.at[0], acc.at[ix2.at[0]], add=True)

            @pl.when(j < n2 - 1)
            def _():
                load(c0 + 2, 0)

            wait_load(c0 + 1, 1)
            pltpu.sync_copy(rw2.at[1], acc.at[ix2.at[1]], add=True)
            return 0

        lax.fori_loop(0, n2, body, 0, unroll=False)
        plsc.subcore_barrier()

        for coff, csz in chunks:
            pltpu.sync_copy(acc.at[pl.ds(row0 + coff, csz)],
                            rw2.at[0, pl.ds(0, csz)])
            pltpu.sync_copy(rw2.at[0, pl.ds(0, csz)],
                            out_h.at[cid, pl.ds(row0 + coff, csz)])

    return k(msg, r_idx)


def kernel(x, node_attr, edge_attr, edge_index, graph_idx, W_embed, W_msg0,
           W_msg1, W_upd0, W_upd1, W_pre0, W_pre1, W_post0, W_out):
    n, d = x.shape
    e = edge_index.shape[1]
    nc, ns = _sc_counts()
    nw = nc * ns
    quant = nw * _CHUNK
    eh = e // 2
    ep = ((eh + quant - 1) // quant) * quant
    pad = ep - eh
    npad = ((n + 1 + 127) // 128) * 128
    np_ = npad - n

    senders = edge_index[0].astype(jnp.int32)
    receivers = edge_index[1].astype(jnp.int32)
    zpad = jnp.zeros((pad,), jnp.int32)
    npd = jnp.full((pad,), n, jnp.int32)
    eapd = jnp.zeros((pad, A), jnp.float32)
    s_h, r_h, rs_h, ea_h = [], [], [], []
    for h in range(2):
        sl = slice(h * eh, (h + 1) * eh)
        s_h.append(jnp.concatenate([senders[sl], zpad]))
        r_h.append(jnp.concatenate([receivers[sl], zpad]))
        rs_h.append(jnp.concatenate([receivers[sl], npd]))
        ea_h.append(jnp.concatenate([edge_attr[sl], eapd], axis=0))
    x_p = jnp.concatenate([x, jnp.zeros((np_, d), jnp.float32)])
    nattr_p = jnp.concatenate([node_attr, jnp.zeros((np_, A), jnp.float32)])

    w_embed_r = W_embed.reshape(D, A * D)
    nodes = _embed(x_p, nattr_p, w_embed_r)

    num_layers = W_msg0.shape[0]
    for l in range(num_layers):
        w0a = W_msg0[l, :D].reshape(D, A * D)
        w0b = W_msg0[l, D:].reshape(D, A * D)
        w1 = W_msg1[l].reshape(D, A * D)
        u0a = W_upd0[l, :D].reshape(D, A * D)
        u0b = W_upd0[l, D:].reshape(D, A * D)
        u1 = W_upd1[l].reshape(D, A * D)

        inc_a, outg_a = _sc_gather(nodes, s_h[0], r_h[0])
        inc_b, outg_b = _sc_gather(nodes, s_h[1], r_h[1])
        msg_a = _edge_mlp(inc_a, outg_a, ea_h[0], w0a, w0b, w1)
        msg_b = _edge_mlp(inc_b, outg_b, ea_h[1], w0a, w0b, w1)
        agg_a = _sc_scatter(msg_a, rs_h[0], npad)
        agg_b = _sc_scatter(msg_b, rs_h[1], npad)
        nodes = _update(nodes, agg_a, agg_b, nattr_p, u0a, u0b, u1)

    h = _prepool(nodes, nattr_p, W_pre0.reshape(D, A * D),
                 W_pre1.reshape(D, A * D))
    gi_p = jnp.concatenate(
        [graph_idx.astype(jnp.int32), jnp.full((np_,), G, jnp.int32)])
    out = _pool_decode(h, gi_p.reshape(npad, 1), W_post0, W_out)
    return out.reshape(G)
